# baseline (speedup 1.0000x reference)
import functools

import jax
import jax.numpy as jnp
from jax import lax
from jax.experimental import pallas as pl
from jax.experimental.pallas import tpu as pltpu
from jax.experimental.pallas import tpu_sc as plsc

F32 = jnp.float32

NB = 4000
EB = 4000
CH = 128


def _silu(v):
    return v * jax.nn.sigmoid(v)


def _pre_body(pos_ref, na_ref, pw_ref, ew_ref, eb_ref, x_ref, h_ref):
    x_ref[...] = jnp.dot(pos_ref[...], pw_ref[...], preferred_element_type=F32)
    h_ref[...] = jnp.dot(na_ref[...], ew_ref[...], preferred_element_type=F32) + eb_ref[...]


def _uv_body(h_ref, pos_ref, wa_ref, wb_ref, we_ref, u_ref, v_ref):
    pe = jnp.dot(pos_ref[...], we_ref[...], preferred_element_type=F32)
    u_ref[...] = jnp.dot(h_ref[...], wa_ref[...], preferred_element_type=F32) + pe
    v_ref[...] = jnp.dot(h_ref[...], wb_ref[...], preferred_element_type=F32) - pe


def _mlp_body(gu_ref, gv_ref, gxr_ref, gxc_ref, wr_ref, b1_ref, w2_ref, b2_ref,
              wc1_ref, bc1_ref, wc2_ref, ef_ref, tr_ref, *, aug):
    cd = gxr_ref[...] - gxc_ref[...]
    rad = jnp.sum(cd * cd, axis=1, keepdims=True)
    pre1 = gu_ref[...] + gv_ref[...] + rad * wr_ref[...] + b1_ref[...]
    t = _silu(pre1)
    ef = _silu(jnp.dot(t, w2_ref[...], preferred_element_type=F32) + b2_ref[...])
    s = _silu(jnp.dot(ef, wc1_ref[...], preferred_element_type=F32) + bc1_ref[...])
    cm = jnp.sum(s * wc2_ref[...], axis=1, keepdims=True)
    tr = cd * cm
    ef_ref[...] = ef
    if aug:
        n = tr.shape[0]
        tr_ref[...] = jnp.concatenate(
            [tr, jnp.ones((n, 1), F32), jnp.zeros((n, 15), F32)], axis=1)
    else:
        tr_ref[...] = tr


def _node_body(x_ref, xagg_ref, deg_ref, h_ref, hagg_ref, wa_ref, wb_ref,
               b1_ref, w2_ref, b2_ref, xo_ref, ho_ref):
    xo_ref[...] = x_ref[...] + xagg_ref[...] / jnp.maximum(deg_ref[...], 1.0)
    nf = _silu(jnp.dot(h_ref[...], wa_ref[...], preferred_element_type=F32)
               + jnp.dot(hagg_ref[...], wb_ref[...], preferred_element_type=F32)
               + b1_ref[...])
    nf = jnp.dot(nf, w2_ref[...], preferred_element_type=F32) + b2_ref[...]
    ho_ref[...] = h_ref[...] + nf


def _out_body(x_ref, w_ref, o_ref):
    o_ref[...] = jnp.dot(x_ref[...], w_ref[...], preferred_element_type=F32)


def _node_spec(d):
    return pl.BlockSpec((NB, d), lambda i: (i, 0))


def _edge_spec(d):
    return pl.BlockSpec((EB, d), lambda i: (i, 0))


def _full_spec(r, c):
    return pl.BlockSpec((r, c), lambda i: (0, 0))


def _sc_gather_body(u_hbm, v_hbm, x_hbm, row_hbm, col_hbm,
                    gu_hbm, gv_hbm, gxr_hbm, gxc_hbm,
                    row_v, col_v, u_v, v_v, xr_v, xc_v, sem_g, sem_s,
                    *, nchunks):
    cid = lax.axis_index("c")
    sid = lax.axis_index("s")
    wid = sid * 2 + cid

    @pl.loop(wid, nchunks, step=32)
    def _chunk(c):
        base = pl.multiple_of(c * CH, CH)
        pltpu.sync_copy(row_hbm.at[pl.ds(base, CH)], row_v)
        pltpu.sync_copy(col_hbm.at[pl.ds(base, CH)], col_v)
        d1 = pltpu.async_copy(u_hbm.at[row_v], u_v, sem_g)
        d2 = pltpu.async_copy(v_hbm.at[col_v], v_v, sem_g)
        d3 = pltpu.async_copy(x_hbm.at[row_v], xr_v, sem_g)
        d4 = pltpu.async_copy(x_hbm.at[col_v], xc_v, sem_g)
        d1.wait(); d2.wait(); d3.wait(); d4.wait()
        s1 = pltpu.async_copy(u_v, gu_hbm.at[pl.ds(base, CH)], sem_s)
        s2 = pltpu.async_copy(v_v, gv_hbm.at[pl.ds(base, CH)], sem_s)
        s3 = pltpu.async_copy(xr_v, gxr_hbm.at[pl.ds(base, CH)], sem_s)
        s4 = pltpu.async_copy(xc_v, gxc_hbm.at[pl.ds(base, CH)], sem_s)
        s1.wait(); s2.wait(); s3.wait(); s4.wait()


def _sc_gather(U, V, x, row, col):
    E = row.shape[0]
    mesh = plsc.VectorSubcoreMesh(core_axis_name="c", subcore_axis_name="s")
    out_type = [
        jax.ShapeDtypeStruct((E, 32), F32),
        jax.ShapeDtypeStruct((E, 32), F32),
        jax.ShapeDtypeStruct((E, 16), F32),
        jax.ShapeDtypeStruct((E, 16), F32),
    ]
    scratch = [
        pltpu.VMEM((CH,), jnp.int32),
        pltpu.VMEM((CH,), jnp.int32),
        pltpu.VMEM((CH, 32), F32),
        pltpu.VMEM((CH, 32), F32),
        pltpu.VMEM((CH, 16), F32),
        pltpu.VMEM((CH, 16), F32),
        pltpu.SemaphoreType.DMA,
        pltpu.SemaphoreType.DMA,
    ]
    fn = pl.kernel(
        functools.partial(_sc_gather_body, nchunks=E // CH),
        out_type=out_type, mesh=mesh, scratch_types=scratch,
        compiler_params=pltpu.CompilerParams(use_tc_tiling_on_sc=False),
        name="egnn_sc_gather")
    return fn(U, V, x, row, col)


_ACC_ROWS = 51200
_ROWS_PER_TILE = _ACC_ROWS // 16
_ZC = 400


def _sc_scatter_body(row_hbm, dat_hbm, out_hbm, acc, row_v, idx_v, dat_v, tmp_v,
                     *, nchunks, n_half, width):
    cid = lax.axis_index("c")
    sid = lax.axis_index("s")
    base_n = cid * n_half

    @pl.loop(0, _ZC)
    def _zrow(i):
        for j in range(width // 16):
            tmp_v[i, pl.ds(j * 16, 16)] = jnp.zeros((16,), F32)

    tile_row0 = sid * _ROWS_PER_TILE

    @pl.loop(0, _ROWS_PER_TILE // _ZC)
    def _zchunk(k):
        off = pl.multiple_of(tile_row0 + k * _ZC, 8)
        pltpu.sync_copy(tmp_v, acc.at[pl.ds(off, _ZC)])

    plsc.subcore_barrier()

    @pl.loop(sid, nchunks, step=16)
    def _chunk(c):
        base = pl.multiple_of(c * CH, CH)
        pltpu.sync_copy(row_hbm.at[pl.ds(base, CH)], row_v)
        pltpu.sync_copy(dat_hbm.at[pl.ds(base, CH)], dat_v)
        for i in range(CH // 16):
            r = row_v[pl.ds(i * 16, 16)]
            ok = (r >= base_n) & (r < base_n + n_half)
            idx_v[pl.ds(i * 16, 16)] = jnp.where(ok, r - base_n, n_half)
        pltpu.sync_copy(dat_v, acc.at[idx_v], add=True)

    plsc.subcore_barrier()

    nreal = jnp.minimum(_ROWS_PER_TILE, jnp.maximum(0, n_half - tile_row0))

    @pl.loop(0, nreal // _ZC)
    def _wo(k):
        off = pl.multiple_of(tile_row0 + k * _ZC, 8)
        pltpu.sync_copy(acc.at[pl.ds(off, _ZC)], tmp_v)
        pltpu.sync_copy(tmp_v, out_hbm.at[pl.ds(base_n + off, _ZC)])


def _sc_scatter(row, dat, n_nodes):
    E, width = dat.shape
    n_half = n_nodes // 2
    mesh = plsc.VectorSubcoreMesh(core_axis_name="c", subcore_axis_name="s")
    scratch = [
        pltpu.VMEM_SHARED((_ACC_ROWS, width), F32),
        pltpu.VMEM((CH,), jnp.int32),
        pltpu.VMEM((CH,), jnp.int32),
        pltpu.VMEM((CH, width), F32),
        pltpu.VMEM((_ZC, width), F32),
    ]
    fn = pl.kernel(
        functools.partial(_sc_scatter_body, nchunks=E // CH,
                          n_half=n_half, width=width),
        out_type=jax.ShapeDtypeStruct((n_nodes, width), F32),
        mesh=mesh, scratch_types=scratch,
        compiler_params=pltpu.CompilerParams(use_tc_tiling_on_sc=False),
        name=f"egnn_sc_scatter{width}")
    return fn(row, dat)


def kernel(node_attrs, positions, edge_index, params):
    row, col = edge_index[0], edge_index[1]
    N = node_attrs.shape[0]
    E = row.shape[0]

    x, h = pl.pallas_call(
        _pre_body,
        grid=(N // NB,),
        in_specs=[_node_spec(3), _node_spec(3), _full_spec(3, 16),
                  _full_spec(3, 32), _full_spec(1, 32)],
        out_specs=[_node_spec(16), _node_spec(32)],
        out_shape=[jax.ShapeDtypeStruct((N, 16), F32),
                   jax.ShapeDtypeStruct((N, 32), F32)],
    )(positions, node_attrs, params["proj_w"].T, params["emb_in_w"].T,
      params["emb_in_b"][None, :])

    deg = None
    for l in range(len(params["layers"])):
        lp = params["layers"][l]
        W1 = lp["edge_w1"]
        wa, wb = W1[:, :32].T, W1[:, 32:64].T
        wr = W1[:, 64][None, :]
        we = W1[:, 65:68].T

        U, V = pl.pallas_call(
            _uv_body,
            grid=(N // NB,),
            in_specs=[_node_spec(32), _node_spec(3), _full_spec(32, 32),
                      _full_spec(32, 32), _full_spec(3, 32)],
            out_specs=[_node_spec(32), _node_spec(32)],
            out_shape=[jax.ShapeDtypeStruct((N, 32), F32),
                       jax.ShapeDtypeStruct((N, 32), F32)],
        )(h, positions, wa, wb, we)

        gu, gv, gxr, gxc = _sc_gather(U, V, x, row, col)

        aug = l == 0
        tw = 32 if aug else 16
        ef, tr = pl.pallas_call(
            functools.partial(_mlp_body, aug=aug),
            grid=(E // EB,),
            in_specs=[_edge_spec(32), _edge_spec(32), _edge_spec(16),
                      _edge_spec(16), _full_spec(1, 32), _full_spec(1, 32),
                      _full_spec(32, 32), _full_spec(1, 32), _full_spec(32, 32),
                      _full_spec(1, 32), _full_spec(1, 32)],
            out_specs=[_edge_spec(32), _edge_spec(tw)],
            out_shape=[jax.ShapeDtypeStruct((E, 32), F32),
                       jax.ShapeDtypeStruct((E, tw), F32)],
        )(gu, gv, gxr, gxc, wr, lp["edge_b1"][None, :], lp["edge_w2"].T,
          lp["edge_b2"][None, :], lp["coord_w1"].T, lp["coord_b1"][None, :],
          lp["coord_w2"])

        hagg = _sc_scatter(row, ef, N)
        ta = _sc_scatter(row, tr, N)
        if aug:
            xagg = ta[:, :16]
            deg = ta[:, 16:17]
        else:
            xagg = ta

        x, h = pl.pallas_call(
            _node_body,
            grid=(N // NB,),
            in_specs=[_node_spec(16), _node_spec(16), _node_spec(1),
                      _node_spec(32), _node_spec(32), _full_spec(32, 32),
                      _full_spec(32, 32), _full_spec(1, 32), _full_spec(32, 32),
                      _full_spec(1, 32)],
            out_specs=[_node_spec(16), _node_spec(32)],
            out_shape=[jax.ShapeDtypeStruct((N, 16), F32),
                       jax.ShapeDtypeStruct((N, 32), F32)],
        )(x, xagg, deg, h, hagg, lp["node_w1"][:, :32].T, lp["node_w1"][:, 32:].T,
          lp["node_b1"][None, :], lp["node_w2"].T, lp["node_b2"][None, :])

    pred = pl.pallas_call(
        _out_body,
        grid=(N // NB,),
        in_specs=[_node_spec(16), _full_spec(16, 3)],
        out_specs=_node_spec(3),
        out_shape=jax.ShapeDtypeStruct((N, 3), F32),
    )(x, params["out_w"].T)
    return pred
